# Initial kernel scaffold; baseline (speedup 1.0000x reference)
import jax
import jax.numpy as jnp
from jax.experimental import pallas as pl


def kernel(node_types, emb, pe, Uq, Uk, qw, kw, vw, qpw, kpw, vpw, sibWo, parWo, qb, kb, vb, qpb, kpb, vpb, lng, lnb, fw1, fb1, fw2, fb2, tdg, tdb, tw1, tb1, tw2, tb2, gw, gb, cw, cb):
    raise NotImplementedError("write your pallas kernel here")


# speedup vs baseline: 3.3138x; 3.3138x over previous
import jax
import jax.numpy as jnp
from jax import lax
from jax.experimental import pallas as pl
from jax.experimental.pallas import tpu as pltpu

N = 87381
D = 128
H = 8
DK = D // H
FAN = 4
DH = 128
NCLS = 104
VOCAB = 1000
LEVELS = 9
LEVEL_START = [(4 ** l - 1) // 3 for l in range(LEVELS + 1)]
EMB_BLOCK = 512
NPAD = ((N + EMB_BLOCK - 1) // EMB_BLOCK) * EMB_BLOCK
POOL_BLOCK = 512
BU_BP = 256
TD_BP = 512
SCALE = 1.0 / (DK ** 0.5)


def _head_mats():
    dh = lax.broadcasted_iota(jnp.int32, (D, H), 0) // DK
    hh = lax.broadcasted_iota(jnp.int32, (D, H), 1)
    sh = (dh == hh).astype(jnp.float32)
    d2 = lax.broadcasted_iota(jnp.int32, (H, D), 1) // DK
    h2 = lax.broadcasted_iota(jnp.int32, (H, D), 0)
    sht = (d2 == h2).astype(jnp.float32)
    return sh, sht


def _ln(x, g, b):
    m = jnp.mean(x, axis=-1, keepdims=True)
    xc = x - m
    v = jnp.mean(xc * xc, axis=-1, keepdims=True)
    return xc * lax.rsqrt(v + 1e-5) * g + b


def _ffn(x, w1, b1, w2, b2):
    return jax.nn.gelu(x @ w1 + b1) @ w2 + b2 + x


def _pe_body(pe4_ref, uq_ref, uk_ref, out_ref):
    pel = pe4_ref[...]
    peq = pel @ uq_ref[...]
    pek = pel @ uk_ref[...]
    sh, _ = _head_mats()
    for i in range(FAN):
        for j in range(FAN):
            r = jnp.dot(peq[i:i + 1, :] * pek[j:j + 1, :], sh,
                        preferred_element_type=jnp.float32) * SCALE
            out_ref[i * FAN + j:i * FAN + j + 1, :] = r


def _embed_body(idx_ref, emb_ref, out_ref):
    b = idx_ref.shape[0]
    idx = idx_ref[...]
    iota = lax.broadcasted_iota(jnp.int32, (b, VOCAB), 1)
    oh = (iota == idx).astype(jnp.float32)
    out_ref[...] = jnp.dot(oh, emb_ref[...], preferred_element_type=jnp.float32)


def _bu_body(hc_ref, hp_ref, pe_ref, qw_ref, kw_ref, vw_ref, sibwo_ref,
             kpw_ref, vpw_ref, qpw_ref, parwo_ref, qb_ref, kb_ref, vb_ref,
             kpb_ref, vpb_ref, qpb_ref, lng_ref, lnb_ref, fw1_ref, fb1_ref,
             fw2_ref, fb2_ref, out_ref):
    bp = hp_ref.shape[0]
    x = hc_ref[...]
    q = x @ qw_ref[...] + qb_ref[...]
    k = x @ kw_ref[...] + kb_ref[...]
    v = x @ vw_ref[...] + vb_ref[...]
    sh, sht = _head_mats()
    pe = pe_ref[...]
    q3 = q.reshape(bp, FAN, D)
    k3 = k.reshape(bp, FAN, D)
    v3 = v.reshape(bp, FAN, D)
    x3 = x.reshape(bp, FAN, D)
    lng = lng_ref[...]
    lnb = lnb_ref[...]
    x2 = []
    for i in range(FAN):
        qi = q3[:, i, :]
        s = [jnp.dot(qi * k3[:, j, :], sh,
                     preferred_element_type=jnp.float32) * SCALE
             + pe[i * FAN + j:i * FAN + j + 1, :] for j in range(FAN)]
        m = jnp.maximum(jnp.maximum(s[0], s[1]), jnp.maximum(s[2], s[3]))
        e = [jnp.exp(sj - m) for sj in s]
        den = e[0] + e[1] + e[2] + e[3]
        oi = sum(jnp.dot(e[j] / den, sht,
                         preferred_element_type=jnp.float32) * v3[:, j, :]
                 for j in range(FAN))
        sai = oi @ sibwo_ref[...]
        x2.append(_ln(sai + x3[:, i, :], lng, lnb))
    nk = [x2i @ kpw_ref[...] + kpb_ref[...] for x2i in x2]
    nv = [x2i @ vpw_ref[...] + vpb_ref[...] for x2i in x2]
    pq = hp_ref[...] @ qpw_ref[...] + qpb_ref[...]
    t = [jnp.dot(pq * nk[j], sh, preferred_element_type=jnp.float32) * SCALE
         for j in range(FAN)]
    m = jnp.maximum(jnp.maximum(t[0], t[1]), jnp.maximum(t[2], t[3]))
    e = [jnp.exp(tj - m) for tj in t]
    den = e[0] + e[1] + e[2] + e[3]
    po = sum(jnp.dot(e[j] / den, sht,
                     preferred_element_type=jnp.float32) * nv[j]
             for j in range(FAN))
    pc = po @ parwo_ref[...]
    pc = _ln(pc + pq, lng, lnb)
    hp_new = _ffn(pc, fw1_ref[...], fb1_ref[...], fw2_ref[...], fb2_ref[...])
    out_ref[...] = _ln(hp_new, lng, lnb)


def _td_body(hp_ref, hc_ref, tdg_ref, tdb_ref, tw1_ref, tb1_ref, tw2_ref,
             tb2_ref, out_ref):
    bp = hp_ref.shape[0]
    hp = hp_ref[...]
    x = hc_ref[...]
    par = jnp.broadcast_to(hp[:, None, :], (bp, FAN, D)).reshape(FAN * bp, D)
    cc = _ln(par + x, tdg_ref[...], tdb_ref[...])
    hc = _ffn(cc, tw1_ref[...], tb1_ref[...], tw2_ref[...], tb2_ref[...])
    out_ref[...] = _ln(hc, tdg_ref[...], tdb_ref[...])


def _pool_body(hb_ref, gw_ref, gb_ref, cw_ref, cb_ref, out_ref,
               m_ref, s_ref, acc_ref):
    i = pl.program_id(0)
    nb = pl.num_programs(0)
    b = hb_ref.shape[0]

    @pl.when(i == 0)
    def _():
        m_ref[0, 0] = -1e30
        s_ref[0, 0] = 0.0
        acc_ref[...] = jnp.zeros_like(acc_ref)

    hb = hb_ref[...]
    z = jnp.dot(hb, gw_ref[...], preferred_element_type=jnp.float32) + gb_ref[0, 0]
    row = i * b + lax.broadcasted_iota(jnp.int32, (b, 1), 0)
    z = jnp.where(row < N, z, -1e30)
    mb = jnp.max(z)
    m_old = m_ref[0, 0]
    m_new = jnp.maximum(m_old, mb)
    c = jnp.exp(m_old - m_new)
    e = jnp.exp(z - m_new)
    m_ref[0, 0] = m_new
    s_ref[0, 0] = s_ref[0, 0] * c + jnp.sum(e)
    acc_ref[...] = acc_ref[...] * c + jnp.sum(e * hb, axis=0, keepdims=True)

    @pl.when(i == nb - 1)
    def _():
        pooled = acc_ref[...] / s_ref[0, 0]
        out_ref[...] = jnp.dot(pooled, cw_ref[...],
                               preferred_element_type=jnp.float32) + cb_ref[...]


def kernel(node_types, emb, pe, Uq, Uk, qw, kw, vw, qpw, kpw, vpw, sibWo,
           parWo, qb, kb, vb, qpb, kpb, vpb, lng, lnb, fw1, fb1, fw2, fb2,
           tdg, tdb, tw1, tb1, tw2, tb2, gw, gb, cw, cb):
    f32 = jnp.float32
    r2 = lambda a: a.reshape(1, -1).astype(f32)
    qb2, kb2, vb2, qpb2, kpb2, vpb2 = map(r2, (qb, kb, vb, qpb, kpb, vpb))
    lng2, lnb2, fb12, fb22 = map(r2, (lng, lnb, fb1, fb2))
    tdg2, tdb2, tb12, tb22 = map(r2, (tdg, tdb, tb1, tb2))
    gb2, cb2 = map(r2, (gb, cb))

    pe_bias = pl.pallas_call(
        _pe_body,
        out_shape=jax.ShapeDtypeStruct((FAN * FAN, H), f32),
    )(pe[:FAN].astype(f32), Uq, Uk)

    idx = jnp.pad(node_types.astype(jnp.int32), (0, NPAD - N)).reshape(NPAD, 1)
    h0 = pl.pallas_call(
        _embed_body,
        grid=(NPAD // EMB_BLOCK,),
        in_specs=[pl.BlockSpec((EMB_BLOCK, 1), lambda i: (i, 0)),
                  pl.BlockSpec((VOCAB, D), lambda i: (0, 0))],
        out_specs=pl.BlockSpec((EMB_BLOCK, D), lambda i: (i, 0)),
        out_shape=jax.ShapeDtypeStruct((NPAD, D), f32),
    )(idx, emb.astype(f32))
    hs = [h0[LEVEL_START[l]:LEVEL_START[l + 1]] for l in range(LEVELS)]

    bu_w = (qw, kw, vw, sibWo, kpw, vpw, qpw, parWo, qb2, kb2, vb2, kpb2,
            vpb2, qpb2, lng2, lnb2, fw1, fb12, fw2, fb22)
    for l in range(LEVELS - 2, -1, -1):
        p = 4 ** l
        bp = min(p, BU_BP)
        specs = [pl.BlockSpec((FAN * bp, D), lambda i: (i, 0)),
                 pl.BlockSpec((bp, D), lambda i: (i, 0)),
                 pl.BlockSpec((FAN * FAN, H), lambda i: (0, 0))]
        specs += [pl.BlockSpec(w.shape, lambda i: (0, 0)) for w in bu_w]
        hs[l] = pl.pallas_call(
            _bu_body,
            grid=(p // bp,),
            in_specs=specs,
            out_specs=pl.BlockSpec((bp, D), lambda i: (i, 0)),
            out_shape=jax.ShapeDtypeStruct((p, D), f32),
        )(hs[l + 1], hs[l], pe_bias, *bu_w)

    td_w = (tdg2, tdb2, tw1, tb12, tw2, tb22)
    for l in range(LEVELS - 1):
        p = 4 ** l
        bp = min(p, TD_BP)
        specs = [pl.BlockSpec((bp, D), lambda i: (i, 0)),
                 pl.BlockSpec((FAN * bp, D), lambda i: (i, 0))]
        specs += [pl.BlockSpec(w.shape, lambda i: (0, 0)) for w in td_w]
        hs[l + 1] = pl.pallas_call(
            _td_body,
            grid=(p // bp,),
            in_specs=specs,
            out_specs=pl.BlockSpec((FAN * bp, D), lambda i: (i, 0)),
            out_shape=jax.ShapeDtypeStruct((FAN * p, D), f32),
        )(hs[l], hs[l + 1], *td_w)

    hbig = jnp.pad(jnp.concatenate(hs, axis=0), ((0, NPAD - N), (0, 0)))
    out = pl.pallas_call(
        _pool_body,
        grid=(NPAD // POOL_BLOCK,),
        in_specs=[pl.BlockSpec((POOL_BLOCK, D), lambda i: (i, 0)),
                  pl.BlockSpec((D, 1), lambda i: (0, 0)),
                  pl.BlockSpec((1, 1), lambda i: (0, 0)),
                  pl.BlockSpec((D, NCLS), lambda i: (0, 0)),
                  pl.BlockSpec((1, NCLS), lambda i: (0, 0))],
        out_specs=pl.BlockSpec((1, NCLS), lambda i: (0, 0)),
        out_shape=jax.ShapeDtypeStruct((1, NCLS), f32),
        scratch_shapes=[pltpu.SMEM((1, 1), f32), pltpu.SMEM((1, 1), f32),
                        pltpu.VMEM((1, D), f32)],
    )(hbig, gw.astype(f32), gb2, cw.astype(f32), cb2)
    return out


# speedup vs baseline: 3.3149x; 1.0003x over previous
import jax
import jax.numpy as jnp
from jax import lax
from jax.experimental import pallas as pl
from jax.experimental.pallas import tpu as pltpu

N = 87381
D = 128
H = 8
DK = D // H
FAN = 4
DH = 128
NCLS = 104
VOCAB = 1000
LEVELS = 9
LEVEL_START = [(4 ** l - 1) // 3 for l in range(LEVELS + 1)]
EMB_BLOCK = 512
NPAD = ((N + EMB_BLOCK - 1) // EMB_BLOCK) * EMB_BLOCK
POOL_BLOCK = 512
BU_BP = 256
TD_BP = 512
SCALE = 1.0 / (DK ** 0.5)


def _head_mats():
    dh = lax.broadcasted_iota(jnp.int32, (D, H), 0) // DK
    hh = lax.broadcasted_iota(jnp.int32, (D, H), 1)
    sh = (dh == hh).astype(jnp.float32)
    d2 = lax.broadcasted_iota(jnp.int32, (H, D), 1) // DK
    h2 = lax.broadcasted_iota(jnp.int32, (H, D), 0)
    sht = (d2 == h2).astype(jnp.float32)
    return sh, sht


def _ln(x, g, b):
    m = jnp.mean(x, axis=-1, keepdims=True)
    xc = x - m
    v = jnp.mean(xc * xc, axis=-1, keepdims=True)
    return xc * lax.rsqrt(v + 1e-5) * g + b


def _ffn(x, w1, b1, w2, b2):
    return jax.nn.gelu(x @ w1 + b1) @ w2 + b2 + x


def _pe_body(pe4_ref, uq_ref, uk_ref, out_ref):
    pel = pe4_ref[...]
    peq = pel @ uq_ref[...]
    pek = pel @ uk_ref[...]
    sh, _ = _head_mats()
    for i in range(FAN):
        for j in range(FAN):
            r = jnp.dot(peq[i:i + 1, :] * pek[j:j + 1, :], sh,
                        preferred_element_type=jnp.float32) * SCALE
            out_ref[i * FAN + j:i * FAN + j + 1, :] = r


def _embed_body(idx_ref, emb_ref, out_ref):
    b = idx_ref.shape[0]
    idx = idx_ref[...]
    iota = lax.broadcasted_iota(jnp.int32, (b, VOCAB), 1)
    oh = (iota == idx).astype(jnp.float32)
    out_ref[...] = jnp.dot(oh, emb_ref[...], preferred_element_type=jnp.float32)


def _bu_body(hc_ref, hp_ref, pe_ref, qw_ref, kw_ref, vw_ref, sibwo_ref,
             kpw_ref, vpw_ref, qpw_ref, parwo_ref, qb_ref, kb_ref, vb_ref,
             kpb_ref, vpb_ref, qpb_ref, lng_ref, lnb_ref, fw1_ref, fb1_ref,
             fw2_ref, fb2_ref, out_ref):
    bp = hp_ref.shape[0]
    x = hc_ref[...]
    q = x @ qw_ref[...] + qb_ref[...]
    k = x @ kw_ref[...] + kb_ref[...]
    v = x @ vw_ref[...] + vb_ref[...]
    sh, sht = _head_mats()
    pe = pe_ref[...]
    q3 = q.reshape(bp, FAN, D)
    k3 = k.reshape(bp, FAN, D)
    v3 = v.reshape(bp, FAN, D)
    x3 = x.reshape(bp, FAN, D)
    lng = lng_ref[...]
    lnb = lnb_ref[...]
    x2 = []
    for i in range(FAN):
        qi = q3[:, i, :]
        s = [jnp.dot(qi * k3[:, j, :], sh,
                     preferred_element_type=jnp.float32) * SCALE
             + pe[i * FAN + j:i * FAN + j + 1, :] for j in range(FAN)]
        m = jnp.maximum(jnp.maximum(s[0], s[1]), jnp.maximum(s[2], s[3]))
        e = [jnp.exp(sj - m) for sj in s]
        den = e[0] + e[1] + e[2] + e[3]
        oi = sum(jnp.dot(e[j] / den, sht,
                         preferred_element_type=jnp.float32) * v3[:, j, :]
                 for j in range(FAN))
        sai = oi @ sibwo_ref[...]
        x2.append(_ln(sai + x3[:, i, :], lng, lnb))
    nk = [x2i @ kpw_ref[...] + kpb_ref[...] for x2i in x2]
    nv = [x2i @ vpw_ref[...] + vpb_ref[...] for x2i in x2]
    pq = hp_ref[...] @ qpw_ref[...] + qpb_ref[...]
    t = [jnp.dot(pq * nk[j], sh, preferred_element_type=jnp.float32) * SCALE
         for j in range(FAN)]
    m = jnp.maximum(jnp.maximum(t[0], t[1]), jnp.maximum(t[2], t[3]))
    e = [jnp.exp(tj - m) for tj in t]
    den = e[0] + e[1] + e[2] + e[3]
    po = sum(jnp.dot(e[j] / den, sht,
                     preferred_element_type=jnp.float32) * nv[j]
             for j in range(FAN))
    pc = po @ parwo_ref[...]
    pc = _ln(pc + pq, lng, lnb)
    hp_new = _ffn(pc, fw1_ref[...], fb1_ref[...], fw2_ref[...], fb2_ref[...])
    out_ref[...] = _ln(hp_new, lng, lnb)


def _td_body(hp_ref, hc_ref, tdg_ref, tdb_ref, tw1_ref, tb1_ref, tw2_ref,
             tb2_ref, out_ref):
    bp = hp_ref.shape[0]
    hp = hp_ref[...]
    x = hc_ref[...]
    par = jnp.broadcast_to(hp[:, None, :], (bp, FAN, D)).reshape(FAN * bp, D)
    cc = _ln(par + x, tdg_ref[...], tdb_ref[...])
    hc = _ffn(cc, tw1_ref[...], tb1_ref[...], tw2_ref[...], tb2_ref[...])
    out_ref[...] = _ln(hc, tdg_ref[...], tdb_ref[...])


def _pool_body(hb_ref, gw_ref, gb_ref, cw_ref, cb_ref, out_ref,
               m_ref, s_ref, acc_ref):
    i = pl.program_id(0)
    nb = pl.num_programs(0)
    b = hb_ref.shape[0]

    @pl.when(i == 0)
    def _():
        m_ref[0, 0] = -1e30
        s_ref[0, 0] = 0.0
        acc_ref[...] = jnp.zeros_like(acc_ref)

    hb = hb_ref[...]
    z = jnp.dot(hb, gw_ref[...], preferred_element_type=jnp.float32) + gb_ref[0, 0]
    row = i * b + lax.broadcasted_iota(jnp.int32, (b, 1), 0)
    z = jnp.where(row < N, z, -1e30)
    mb = jnp.max(z)
    m_old = m_ref[0, 0]
    m_new = jnp.maximum(m_old, mb)
    c = jnp.exp(m_old - m_new)
    e = jnp.exp(z - m_new)
    m_ref[0, 0] = m_new
    s_ref[0, 0] = s_ref[0, 0] * c + jnp.sum(e)
    acc_ref[...] = acc_ref[...] * c + jnp.sum(e * hb, axis=0, keepdims=True)

    @pl.when(i == nb - 1)
    def _():
        pooled = acc_ref[...] / s_ref[0, 0]
        out_ref[...] = jnp.dot(pooled, cw_ref[...],
                               preferred_element_type=jnp.float32) + cb_ref[...]


def kernel(node_types, emb, pe, Uq, Uk, qw, kw, vw, qpw, kpw, vpw, sibWo,
           parWo, qb, kb, vb, qpb, kpb, vpb, lng, lnb, fw1, fb1, fw2, fb2,
           tdg, tdb, tw1, tb1, tw2, tb2, gw, gb, cw, cb):
    f32 = jnp.float32
    r2 = lambda a: a.reshape(1, -1).astype(f32)
    qb2, kb2, vb2, qpb2, kpb2, vpb2 = map(r2, (qb, kb, vb, qpb, kpb, vpb))
    lng2, lnb2, fb12, fb22 = map(r2, (lng, lnb, fb1, fb2))
    tdg2, tdb2, tb12, tb22 = map(r2, (tdg, tdb, tb1, tb2))
    gb2, cb2 = map(r2, (gb, cb))

    pe_bias = pl.pallas_call(
        _pe_body,
        out_shape=jax.ShapeDtypeStruct((FAN * FAN, H), f32),
    )(pe[:FAN].astype(f32), Uq, Uk)

    par = pltpu.CompilerParams(dimension_semantics=("parallel",))
    idx = jnp.pad(node_types.astype(jnp.int32), (0, NPAD - N)).reshape(NPAD, 1)
    h0 = pl.pallas_call(
        _embed_body,
        grid=(NPAD // EMB_BLOCK,),
        in_specs=[pl.BlockSpec((EMB_BLOCK, 1), lambda i: (i, 0)),
                  pl.BlockSpec((VOCAB, D), lambda i: (0, 0))],
        out_specs=pl.BlockSpec((EMB_BLOCK, D), lambda i: (i, 0)),
        out_shape=jax.ShapeDtypeStruct((NPAD, D), f32),
        compiler_params=par,
    )(idx, emb.astype(f32))
    hs = [h0[LEVEL_START[l]:LEVEL_START[l + 1]] for l in range(LEVELS)]

    bu_w = (qw, kw, vw, sibWo, kpw, vpw, qpw, parWo, qb2, kb2, vb2, kpb2,
            vpb2, qpb2, lng2, lnb2, fw1, fb12, fw2, fb22)
    for l in range(LEVELS - 2, -1, -1):
        p = 4 ** l
        bp = min(p, BU_BP)
        specs = [pl.BlockSpec((FAN * bp, D), lambda i: (i, 0)),
                 pl.BlockSpec((bp, D), lambda i: (i, 0)),
                 pl.BlockSpec((FAN * FAN, H), lambda i: (0, 0))]
        specs += [pl.BlockSpec(w.shape, lambda i: (0, 0)) for w in bu_w]
        hs[l] = pl.pallas_call(
            _bu_body,
            grid=(p // bp,),
            in_specs=specs,
            out_specs=pl.BlockSpec((bp, D), lambda i: (i, 0)),
            out_shape=jax.ShapeDtypeStruct((p, D), f32),
            compiler_params=par,
        )(hs[l + 1], hs[l], pe_bias, *bu_w)

    td_w = (tdg2, tdb2, tw1, tb12, tw2, tb22)
    for l in range(LEVELS - 1):
        p = 4 ** l
        bp = min(p, TD_BP)
        specs = [pl.BlockSpec((bp, D), lambda i: (i, 0)),
                 pl.BlockSpec((FAN * bp, D), lambda i: (i, 0))]
        specs += [pl.BlockSpec(w.shape, lambda i: (0, 0)) for w in td_w]
        hs[l + 1] = pl.pallas_call(
            _td_body,
            grid=(p // bp,),
            in_specs=specs,
            out_specs=pl.BlockSpec((FAN * bp, D), lambda i: (i, 0)),
            out_shape=jax.ShapeDtypeStruct((FAN * p, D), f32),
            compiler_params=par,
        )(hs[l], hs[l + 1], *td_w)

    hbig = jnp.pad(jnp.concatenate(hs, axis=0), ((0, NPAD - N), (0, 0)))
    out = pl.pallas_call(
        _pool_body,
        grid=(NPAD // POOL_BLOCK,),
        in_specs=[pl.BlockSpec((POOL_BLOCK, D), lambda i: (i, 0)),
                  pl.BlockSpec((D, 1), lambda i: (0, 0)),
                  pl.BlockSpec((1, 1), lambda i: (0, 0)),
                  pl.BlockSpec((D, NCLS), lambda i: (0, 0)),
                  pl.BlockSpec((1, NCLS), lambda i: (0, 0))],
        out_specs=pl.BlockSpec((1, NCLS), lambda i: (0, 0)),
        out_shape=jax.ShapeDtypeStruct((1, NCLS), f32),
        scratch_shapes=[pltpu.SMEM((1, 1), f32), pltpu.SMEM((1, 1), f32),
                        pltpu.VMEM((1, D), f32)],
    )(hbig, gw.astype(f32), gb2, cw.astype(f32), cb2)
    return out


# speedup vs baseline: 3.5670x; 1.0761x over previous
import jax
import jax.numpy as jnp
from jax import lax
from jax.experimental import pallas as pl
from jax.experimental.pallas import tpu as pltpu

N = 87381
D = 128
H = 8
DK = D // H
FAN = 4
DH = 128
NCLS = 104
VOCAB = 1000
LEVELS = 9
LEVEL_START = [(4 ** l - 1) // 3 for l in range(LEVELS + 1)]
EMB_BLOCK = 512
POOL_BLOCK = 4096
NPAD = ((N + POOL_BLOCK - 1) // POOL_BLOCK) * POOL_BLOCK
BU_BP = 256
TD_BP = 512
SCALE = 1.0 / (DK ** 0.5)


def _head_mats():
    dh = lax.broadcasted_iota(jnp.int32, (D, H), 0) // DK
    hh = lax.broadcasted_iota(jnp.int32, (D, H), 1)
    sh = (dh == hh).astype(jnp.float32)
    d2 = lax.broadcasted_iota(jnp.int32, (H, D), 1) // DK
    h2 = lax.broadcasted_iota(jnp.int32, (H, D), 0)
    sht = (d2 == h2).astype(jnp.float32)
    return sh, sht


def _ln(x, g, b):
    m = jnp.mean(x, axis=-1, keepdims=True)
    xc = x - m
    v = jnp.mean(xc * xc, axis=-1, keepdims=True)
    return xc * lax.rsqrt(v + 1e-5) * g + b


def _ffn(x, w1, b1, w2, b2):
    return jax.nn.gelu(x @ w1 + b1) @ w2 + b2 + x


def _pe_body(pe4_ref, uq_ref, uk_ref, out_ref):
    pel = pe4_ref[...]
    peq = pel @ uq_ref[...]
    pek = pel @ uk_ref[...]
    sh, _ = _head_mats()
    for i in range(FAN):
        for j in range(FAN):
            r = jnp.dot(peq[i:i + 1, :] * pek[j:j + 1, :], sh,
                        preferred_element_type=jnp.float32) * SCALE
            out_ref[i * FAN + j:i * FAN + j + 1, :] = r


def _embed_body(idx_ref, ehi_ref, elo_ref, out_ref):
    b = idx_ref.shape[0]
    idx = idx_ref[...]
    iota = lax.broadcasted_iota(jnp.int32, (b, VOCAB), 1)
    oh = (iota == idx).astype(jnp.bfloat16)
    out_ref[...] = (
        jnp.dot(oh, ehi_ref[...], preferred_element_type=jnp.float32)
        + jnp.dot(oh, elo_ref[...], preferred_element_type=jnp.float32))


def _groll(a, r):
    if r == 0:
        return a
    a4 = a.reshape(a.shape[0] // FAN, FAN, a.shape[1])
    return jnp.concatenate([a4[:, r:, :], a4[:, :r, :]],
                           axis=1).reshape(a.shape)


def _bu_body(hc_ref, hp_ref, pe_ref, qw_ref, kw_ref, vw_ref, sibwo_ref,
             kpw_ref, vpw_ref, qpw_ref, parwo_ref, qb_ref, kb_ref, vb_ref,
             kpb_ref, vpb_ref, qpb_ref, lng_ref, lnb_ref, fw1_ref, fb1_ref,
             fw2_ref, fb2_ref, out_ref):
    bp = hp_ref.shape[0]
    c4 = FAN * bp
    f32 = jnp.float32
    x = hc_ref[...]
    q = x @ qw_ref[...] + qb_ref[...]
    k = x @ kw_ref[...] + kb_ref[...]
    v = x @ vw_ref[...] + vb_ref[...]
    sh, sht = _head_mats()
    pe = pe_ref[...]
    lng = lng_ref[...]
    lnb = lnb_ref[...]
    q3 = q.reshape(bp, FAN, D)
    k3 = k.reshape(bp, FAN, D)
    v3 = v.reshape(bp, FAN, D)
    x3 = x.reshape(bp, FAN, D)
    x2 = []
    for i in range(FAN):
        qi = q3[:, i, :]
        s = [jnp.dot(qi * k3[:, j, :], sh, preferred_element_type=f32)
             * SCALE + pe[i * FAN + j:i * FAN + j + 1, :] for j in range(FAN)]
        m = jnp.maximum(jnp.maximum(s[0], s[1]), jnp.maximum(s[2], s[3]))
        e = [jnp.exp(sj - m) for sj in s]
        den = e[0] + e[1] + e[2] + e[3]
        oi = sum(jnp.dot(e[j] / den, sht, preferred_element_type=f32)
                 * v3[:, j, :] for j in range(FAN))
        sai = oi @ sibwo_ref[...]
        x2.append(_ln(sai + x3[:, i, :], lng, lnb))
    nk = [x2i @ kpw_ref[...] + kpb_ref[...] for x2i in x2]
    nv = [x2i @ vpw_ref[...] + vpb_ref[...] for x2i in x2]
    pq = hp_ref[...] @ qpw_ref[...] + qpb_ref[...]
    t = [jnp.dot(pq * nk[j], sh, preferred_element_type=f32) * SCALE
         for j in range(FAN)]
    m = jnp.maximum(jnp.maximum(t[0], t[1]), jnp.maximum(t[2], t[3]))
    e = [jnp.exp(tj - m) for tj in t]
    den = e[0] + e[1] + e[2] + e[3]
    po = sum(jnp.dot(e[j] / den, sht, preferred_element_type=f32) * nv[j]
             for j in range(FAN))
    pc = po @ parwo_ref[...]
    pc = _ln(pc + pq, lng, lnb)
    hp_new = _ffn(pc, fw1_ref[...], fb1_ref[...], fw2_ref[...], fb2_ref[...])
    out_ref[...] = _ln(hp_new, lng, lnb)


def _td_body(hp_ref, hc_ref, tdg_ref, tdb_ref, tw1_ref, tb1_ref, tw2_ref,
             tb2_ref, out_ref):
    bp = hp_ref.shape[0]
    hp = hp_ref[...]
    x = hc_ref[...]
    par = jnp.broadcast_to(hp[:, None, :], (bp, FAN, D)).reshape(FAN * bp, D)
    cc = _ln(par + x, tdg_ref[...], tdb_ref[...])
    hc = _ffn(cc, tw1_ref[...], tb1_ref[...], tw2_ref[...], tb2_ref[...])
    out_ref[...] = _ln(hc, tdg_ref[...], tdb_ref[...])


def _pool_body(hb_ref, gw_ref, gb_ref, cw_ref, cb_ref, out_ref,
               m_ref, s_ref, acc_ref):
    i = pl.program_id(0)
    nb = pl.num_programs(0)
    b = hb_ref.shape[0]

    @pl.when(i == 0)
    def _():
        m_ref[0, 0] = -1e30
        s_ref[0, 0] = 0.0
        acc_ref[...] = jnp.zeros_like(acc_ref)

    hb = hb_ref[...]
    z = jnp.dot(hb, gw_ref[...], preferred_element_type=jnp.float32) + gb_ref[0, 0]
    row = i * b + lax.broadcasted_iota(jnp.int32, (b, 1), 0)
    z = jnp.where(row < N, z, -1e30)
    mb = jnp.max(z)
    m_old = m_ref[0, 0]
    m_new = jnp.maximum(m_old, mb)
    c = jnp.exp(m_old - m_new)
    e = jnp.exp(z - m_new)
    m_ref[0, 0] = m_new
    s_ref[0, 0] = s_ref[0, 0] * c + jnp.sum(e)
    acc_ref[...] = acc_ref[...] * c + jnp.sum(e * hb, axis=0, keepdims=True)

    @pl.when(i == nb - 1)
    def _():
        pooled = acc_ref[...] / s_ref[0, 0]
        out_ref[...] = jnp.dot(pooled, cw_ref[...],
                               preferred_element_type=jnp.float32) + cb_ref[...]


def kernel(node_types, emb, pe, Uq, Uk, qw, kw, vw, qpw, kpw, vpw, sibWo,
           parWo, qb, kb, vb, qpb, kpb, vpb, lng, lnb, fw1, fb1, fw2, fb2,
           tdg, tdb, tw1, tb1, tw2, tb2, gw, gb, cw, cb):
    f32 = jnp.float32
    r2 = lambda a: a.reshape(1, -1).astype(f32)
    qb2, kb2, vb2, qpb2, kpb2, vpb2 = map(r2, (qb, kb, vb, qpb, kpb, vpb))
    lng2, lnb2, fb12, fb22 = map(r2, (lng, lnb, fb1, fb2))
    tdg2, tdb2, tb12, tb22 = map(r2, (tdg, tdb, tb1, tb2))
    gb2, cb2 = map(r2, (gb, cb))

    pe_bias = pl.pallas_call(
        _pe_body,
        out_shape=jax.ShapeDtypeStruct((FAN * FAN, H), f32),
    )(pe[:FAN].astype(f32), Uq, Uk)

    par = pltpu.CompilerParams(dimension_semantics=("parallel",))
    idx = jnp.pad(node_types.astype(jnp.int32), (0, NPAD - N)).reshape(NPAD, 1)
    emb32 = emb.astype(f32)
    ehi = emb32.astype(jnp.bfloat16)
    elo = (emb32 - ehi.astype(f32)).astype(jnp.bfloat16)
    h0 = pl.pallas_call(
        _embed_body,
        grid=(NPAD // EMB_BLOCK,),
        in_specs=[pl.BlockSpec((EMB_BLOCK, 1), lambda i: (i, 0)),
                  pl.BlockSpec((VOCAB, D), lambda i: (0, 0)),
                  pl.BlockSpec((VOCAB, D), lambda i: (0, 0))],
        out_specs=pl.BlockSpec((EMB_BLOCK, D), lambda i: (i, 0)),
        out_shape=jax.ShapeDtypeStruct((NPAD, D), f32),
        compiler_params=par,
    )(idx, ehi, elo)
    hs = [h0[LEVEL_START[l]:LEVEL_START[l + 1]] for l in range(LEVELS)]

    bu_w = (qw, kw, vw, sibWo, kpw, vpw, qpw, parWo, qb2, kb2, vb2, kpb2,
            vpb2, qpb2, lng2, lnb2, fw1, fb12, fw2, fb22)
    for l in range(LEVELS - 2, -1, -1):
        p = 4 ** l
        bp = min(p, BU_BP)
        specs = [pl.BlockSpec((FAN * bp, D), lambda i: (i, 0)),
                 pl.BlockSpec((bp, D), lambda i: (i, 0)),
                 pl.BlockSpec((FAN * FAN, H), lambda i: (0, 0))]
        specs += [pl.BlockSpec(w.shape, lambda i: (0, 0)) for w in bu_w]
        hs[l] = pl.pallas_call(
            _bu_body,
            grid=(p // bp,),
            in_specs=specs,
            out_specs=pl.BlockSpec((bp, D), lambda i: (i, 0)),
            out_shape=jax.ShapeDtypeStruct((p, D), f32),
            compiler_params=par,
        )(hs[l + 1], hs[l], pe_bias, *bu_w)

    td_w = (tdg2, tdb2, tw1, tb12, tw2, tb22)
    for l in range(LEVELS - 1):
        p = 4 ** l
        bp = min(p, TD_BP)
        specs = [pl.BlockSpec((bp, D), lambda i: (i, 0)),
                 pl.BlockSpec((FAN * bp, D), lambda i: (i, 0))]
        specs += [pl.BlockSpec(w.shape, lambda i: (0, 0)) for w in td_w]
        hs[l + 1] = pl.pallas_call(
            _td_body,
            grid=(p // bp,),
            in_specs=specs,
            out_specs=pl.BlockSpec((FAN * bp, D), lambda i: (i, 0)),
            out_shape=jax.ShapeDtypeStruct((FAN * p, D), f32),
            compiler_params=par,
        )(hs[l], hs[l + 1], *td_w)

    hbig = jnp.pad(jnp.concatenate(hs, axis=0), ((0, NPAD - N), (0, 0)))
    out = pl.pallas_call(
        _pool_body,
        grid=(NPAD // POOL_BLOCK,),
        in_specs=[pl.BlockSpec((POOL_BLOCK, D), lambda i: (i, 0)),
                  pl.BlockSpec((D, 1), lambda i: (0, 0)),
                  pl.BlockSpec((1, 1), lambda i: (0, 0)),
                  pl.BlockSpec((D, NCLS), lambda i: (0, 0)),
                  pl.BlockSpec((1, NCLS), lambda i: (0, 0))],
        out_specs=pl.BlockSpec((1, NCLS), lambda i: (0, 0)),
        out_shape=jax.ShapeDtypeStruct((1, NCLS), f32),
        scratch_shapes=[pltpu.SMEM((1, 1), f32), pltpu.SMEM((1, 1), f32),
                        pltpu.VMEM((1, D), f32)],
    )(hbig, gw.astype(f32), gb2, cw.astype(f32), cb2)
    return out


# speedup vs baseline: 4.0446x; 1.1339x over previous
import jax
import jax.numpy as jnp
from jax import lax
from jax.experimental import pallas as pl
from jax.experimental.pallas import tpu as pltpu

N = 87381
D = 128
H = 8
DK = D // H
FAN = 4
DH = 128
NCLS = 104
VOCAB = 1000
LEVELS = 9
LEVEL_START = [(4 ** l - 1) // 3 for l in range(LEVELS + 1)]
EMB_BLOCK = 1024
POOL_BLOCK = 4096
NPAD = ((N + POOL_BLOCK - 1) // POOL_BLOCK) * POOL_BLOCK
BU_BP = 512
TD_BP = 512
SCALE = 1.0 / (DK ** 0.5)


def _head_mats():
    dh = lax.broadcasted_iota(jnp.int32, (D, H), 0) // DK
    hh = lax.broadcasted_iota(jnp.int32, (D, H), 1)
    sh = (dh == hh).astype(jnp.float32)
    d2 = lax.broadcasted_iota(jnp.int32, (H, D), 1) // DK
    h2 = lax.broadcasted_iota(jnp.int32, (H, D), 0)
    sht = (d2 == h2).astype(jnp.float32)
    return sh, sht


def _ln(x, g, b):
    m = jnp.mean(x, axis=-1, keepdims=True)
    xc = x - m
    v = jnp.mean(xc * xc, axis=-1, keepdims=True)
    return xc * lax.rsqrt(v + 1e-5) * g + b


def _ffn(x, w1, b1, w2, b2):
    return jax.nn.gelu(x @ w1 + b1) @ w2 + b2 + x


def _pe_body(pe4_ref, uq_ref, uk_ref, out_ref):
    pel = pe4_ref[...]
    peq = pel @ uq_ref[...]
    pek = pel @ uk_ref[...]
    sh, _ = _head_mats()
    for i in range(FAN):
        for j in range(FAN):
            r = jnp.dot(peq[i:i + 1, :] * pek[j:j + 1, :], sh,
                        preferred_element_type=jnp.float32) * SCALE
            out_ref[i * FAN + j:i * FAN + j + 1, :] = r


def _embed_body(idx_ref, ehi_ref, elo_ref, out_ref):
    b = idx_ref.shape[0]
    idx = idx_ref[...]
    iota = lax.broadcasted_iota(jnp.int32, (b, VOCAB), 1)
    oh = (iota == idx).astype(jnp.bfloat16)
    out_ref[...] = (
        jnp.dot(oh, ehi_ref[...], preferred_element_type=jnp.float32)
        + jnp.dot(oh, elo_ref[...], preferred_element_type=jnp.float32))


def _groll(a, r):
    if r == 0:
        return a
    a4 = a.reshape(a.shape[0] // FAN, FAN, a.shape[1])
    return jnp.concatenate([a4[:, r:, :], a4[:, :r, :]],
                           axis=1).reshape(a.shape)


def _bu_body(hc_ref, hp_ref, pe_ref, qw_ref, kw_ref, vw_ref, sibwo_ref,
             kpw_ref, vpw_ref, qpw_ref, parwo_ref, qb_ref, kb_ref, vb_ref,
             kpb_ref, vpb_ref, qpb_ref, lng_ref, lnb_ref, fw1_ref, fb1_ref,
             fw2_ref, fb2_ref, out_ref):
    bp = hp_ref.shape[0]
    c4 = FAN * bp
    f32 = jnp.float32
    x = hc_ref[...]
    q = x @ qw_ref[...] + qb_ref[...]
    k = x @ kw_ref[...] + kb_ref[...]
    v = x @ vw_ref[...] + vb_ref[...]
    sh, sht = _head_mats()
    pe = pe_ref[...]
    lng = lng_ref[...]
    lnb = lnb_ref[...]
    q3 = q.reshape(bp, FAN, D)
    k3 = k.reshape(bp, FAN, D)
    v3 = v.reshape(bp, FAN, D)
    x3 = x.reshape(bp, FAN, D)
    x2 = []
    for i in range(FAN):
        qi = q3[:, i, :]
        s = [jnp.dot(qi * k3[:, j, :], sh, preferred_element_type=f32)
             * SCALE + pe[i * FAN + j:i * FAN + j + 1, :] for j in range(FAN)]
        m = jnp.maximum(jnp.maximum(s[0], s[1]), jnp.maximum(s[2], s[3]))
        e = [jnp.exp(sj - m) for sj in s]
        den = e[0] + e[1] + e[2] + e[3]
        oi = sum(jnp.dot(e[j] / den, sht, preferred_element_type=f32)
                 * v3[:, j, :] for j in range(FAN))
        sai = oi @ sibwo_ref[...]
        x2.append(_ln(sai + x3[:, i, :], lng, lnb))
    nk = [x2i @ kpw_ref[...] + kpb_ref[...] for x2i in x2]
    nv = [x2i @ vpw_ref[...] + vpb_ref[...] for x2i in x2]
    pq = hp_ref[...] @ qpw_ref[...] + qpb_ref[...]
    t = [jnp.dot(pq * nk[j], sh, preferred_element_type=f32) * SCALE
         for j in range(FAN)]
    m = jnp.maximum(jnp.maximum(t[0], t[1]), jnp.maximum(t[2], t[3]))
    e = [jnp.exp(tj - m) for tj in t]
    den = e[0] + e[1] + e[2] + e[3]
    po = sum(jnp.dot(e[j] / den, sht, preferred_element_type=f32) * nv[j]
             for j in range(FAN))
    pc = po @ parwo_ref[...]
    pc = _ln(pc + pq, lng, lnb)
    hp_new = _ffn(pc, fw1_ref[...], fb1_ref[...], fw2_ref[...], fb2_ref[...])
    out_ref[...] = _ln(hp_new, lng, lnb)


def _td_body(hp_ref, hc_ref, tdg_ref, tdb_ref, tw1_ref, tb1_ref, tw2_ref,
             tb2_ref, out_ref):
    bp = hp_ref.shape[0]
    hp = hp_ref[...]
    x = hc_ref[...]
    par = jnp.broadcast_to(hp[:, None, :], (bp, FAN, D)).reshape(FAN * bp, D)
    cc = _ln(par + x, tdg_ref[...], tdb_ref[...])
    hc = _ffn(cc, tw1_ref[...], tb1_ref[...], tw2_ref[...], tb2_ref[...])
    out_ref[...] = _ln(hc, tdg_ref[...], tdb_ref[...])


def _pool_body(hb_ref, gw_ref, gb_ref, cw_ref, cb_ref, out_ref,
               m_ref, s_ref, acc_ref):
    i = pl.program_id(0)
    nb = pl.num_programs(0)
    b = hb_ref.shape[0]

    @pl.when(i == 0)
    def _():
        m_ref[0, 0] = -1e30
        s_ref[0, 0] = 0.0
        acc_ref[...] = jnp.zeros_like(acc_ref)

    hb = hb_ref[...]
    z = jnp.dot(hb, gw_ref[...], preferred_element_type=jnp.float32) + gb_ref[0, 0]
    row = i * b + lax.broadcasted_iota(jnp.int32, (b, 1), 0)
    z = jnp.where(row < N, z, -1e30)
    mb = jnp.max(z)
    m_old = m_ref[0, 0]
    m_new = jnp.maximum(m_old, mb)
    c = jnp.exp(m_old - m_new)
    e = jnp.exp(z - m_new)
    m_ref[0, 0] = m_new
    s_ref[0, 0] = s_ref[0, 0] * c + jnp.sum(e)
    acc_ref[...] = acc_ref[...] * c + jnp.sum(e * hb, axis=0, keepdims=True)

    @pl.when(i == nb - 1)
    def _():
        pooled = acc_ref[...] / s_ref[0, 0]
        out_ref[...] = jnp.dot(pooled, cw_ref[...],
                               preferred_element_type=jnp.float32) + cb_ref[...]


def kernel(node_types, emb, pe, Uq, Uk, qw, kw, vw, qpw, kpw, vpw, sibWo,
           parWo, qb, kb, vb, qpb, kpb, vpb, lng, lnb, fw1, fb1, fw2, fb2,
           tdg, tdb, tw1, tb1, tw2, tb2, gw, gb, cw, cb):
    f32 = jnp.float32
    r2 = lambda a: a.reshape(1, -1).astype(f32)
    qb2, kb2, vb2, qpb2, kpb2, vpb2 = map(r2, (qb, kb, vb, qpb, kpb, vpb))
    lng2, lnb2, fb12, fb22 = map(r2, (lng, lnb, fb1, fb2))
    tdg2, tdb2, tb12, tb22 = map(r2, (tdg, tdb, tb1, tb2))
    gb2, cb2 = map(r2, (gb, cb))

    pe_bias = pl.pallas_call(
        _pe_body,
        out_shape=jax.ShapeDtypeStruct((FAN * FAN, H), f32),
    )(pe[:FAN].astype(f32), Uq, Uk)

    par = pltpu.CompilerParams(dimension_semantics=("parallel",))
    idx = jnp.pad(node_types.astype(jnp.int32), (0, NPAD - N)).reshape(NPAD, 1)
    emb32 = emb.astype(f32)
    ehi = emb32.astype(jnp.bfloat16)
    elo = (emb32 - ehi.astype(f32)).astype(jnp.bfloat16)
    h0 = pl.pallas_call(
        _embed_body,
        grid=(NPAD // EMB_BLOCK,),
        in_specs=[pl.BlockSpec((EMB_BLOCK, 1), lambda i: (i, 0)),
                  pl.BlockSpec((VOCAB, D), lambda i: (0, 0)),
                  pl.BlockSpec((VOCAB, D), lambda i: (0, 0))],
        out_specs=pl.BlockSpec((EMB_BLOCK, D), lambda i: (i, 0)),
        out_shape=jax.ShapeDtypeStruct((NPAD, D), f32),
        compiler_params=par,
    )(idx, ehi, elo)
    hs = [h0[LEVEL_START[l]:LEVEL_START[l + 1]] for l in range(LEVELS)]

    bu_w = (qw, kw, vw, sibWo, kpw, vpw, qpw, parWo, qb2, kb2, vb2, kpb2,
            vpb2, qpb2, lng2, lnb2, fw1, fb12, fw2, fb22)
    for l in range(LEVELS - 2, -1, -1):
        p = 4 ** l
        bp = min(p, BU_BP)
        specs = [pl.BlockSpec((FAN * bp, D), lambda i: (i, 0)),
                 pl.BlockSpec((bp, D), lambda i: (i, 0)),
                 pl.BlockSpec((FAN * FAN, H), lambda i: (0, 0))]
        specs += [pl.BlockSpec(w.shape, lambda i: (0, 0)) for w in bu_w]
        hs[l] = pl.pallas_call(
            _bu_body,
            grid=(p // bp,),
            in_specs=specs,
            out_specs=pl.BlockSpec((bp, D), lambda i: (i, 0)),
            out_shape=jax.ShapeDtypeStruct((p, D), f32),
            compiler_params=par,
        )(hs[l + 1], hs[l], pe_bias, *bu_w)

    td_w = (tdg2, tdb2, tw1, tb12, tw2, tb22)
    for l in range(LEVELS - 1):
        p = 4 ** l
        bp = min(p, TD_BP)
        specs = [pl.BlockSpec((bp, D), lambda i: (i, 0)),
                 pl.BlockSpec((FAN * bp, D), lambda i: (i, 0))]
        specs += [pl.BlockSpec(w.shape, lambda i: (0, 0)) for w in td_w]
        hs[l + 1] = pl.pallas_call(
            _td_body,
            grid=(p // bp,),
            in_specs=specs,
            out_specs=pl.BlockSpec((FAN * bp, D), lambda i: (i, 0)),
            out_shape=jax.ShapeDtypeStruct((FAN * p, D), f32),
            compiler_params=par,
        )(hs[l], hs[l + 1], *td_w)

    hbig = jnp.pad(jnp.concatenate(hs, axis=0), ((0, NPAD - N), (0, 0)))
    out = pl.pallas_call(
        _pool_body,
        grid=(NPAD // POOL_BLOCK,),
        in_specs=[pl.BlockSpec((POOL_BLOCK, D), lambda i: (i, 0)),
                  pl.BlockSpec((D, 1), lambda i: (0, 0)),
                  pl.BlockSpec((1, 1), lambda i: (0, 0)),
                  pl.BlockSpec((D, NCLS), lambda i: (0, 0)),
                  pl.BlockSpec((1, NCLS), lambda i: (0, 0))],
        out_specs=pl.BlockSpec((1, NCLS), lambda i: (0, 0)),
        out_shape=jax.ShapeDtypeStruct((1, NCLS), f32),
        scratch_shapes=[pltpu.SMEM((1, 1), f32), pltpu.SMEM((1, 1), f32),
                        pltpu.VMEM((1, D), f32)],
    )(hbig, gw.astype(f32), gb2, cw.astype(f32), cb2)
    return out


# speedup vs baseline: 4.9279x; 1.2184x over previous
import jax
import jax.numpy as jnp
from jax import lax
from jax.experimental import pallas as pl
from jax.experimental.pallas import tpu as pltpu

N = 87381
D = 128
H = 8
DK = D // H
FAN = 4
DH = 128
NCLS = 104
VOCAB = 1000
LEVELS = 9
LEVEL_START = [(4 ** l - 1) // 3 for l in range(LEVELS + 1)]
EMB_BLOCK = 2048
POOL_BLOCK = 4096
NPAD = ((N + POOL_BLOCK - 1) // POOL_BLOCK) * POOL_BLOCK
BU_BP = 512
TD_BP = 1024
SCALE = 1.0 / (DK ** 0.5)


def _head_mats():
    dh = lax.broadcasted_iota(jnp.int32, (D, H), 0) // DK
    hh = lax.broadcasted_iota(jnp.int32, (D, H), 1)
    sh = (dh == hh).astype(jnp.float32)
    d2 = lax.broadcasted_iota(jnp.int32, (H, D), 1) // DK
    h2 = lax.broadcasted_iota(jnp.int32, (H, D), 0)
    sht = (d2 == h2).astype(jnp.float32)
    return sh, sht


def _ln(x, g, b):
    m = jnp.mean(x, axis=-1, keepdims=True)
    xc = x - m
    v = jnp.mean(xc * xc, axis=-1, keepdims=True)
    return xc * lax.rsqrt(v + 1e-5) * g + b


def _ffn(x, w1, b1, w2, b2):
    return jax.nn.gelu(x @ w1 + b1) @ w2 + b2 + x


def _pe_body(pe4_ref, uq_ref, uk_ref, out_ref):
    pel = pe4_ref[...]
    peq = pel @ uq_ref[...]
    pek = pel @ uk_ref[...]
    sh, _ = _head_mats()
    for i in range(FAN):
        for j in range(FAN):
            r = jnp.dot(peq[i:i + 1, :] * pek[j:j + 1, :], sh,
                        preferred_element_type=jnp.float32) * SCALE
            out_ref[i * FAN + j:i * FAN + j + 1, :] = r


def _embed_body(idx_ref, ehi_ref, elo_ref, out_ref):
    b = idx_ref.shape[0]
    idx = idx_ref[...]
    iota = lax.broadcasted_iota(jnp.int32, (b, VOCAB), 1)
    oh = (iota == idx).astype(jnp.bfloat16)
    out_ref[...] = (
        jnp.dot(oh, ehi_ref[...], preferred_element_type=jnp.float32)
        + jnp.dot(oh, elo_ref[...], preferred_element_type=jnp.float32))


def _groll(a, r):
    if r == 0:
        return a
    a4 = a.reshape(a.shape[0] // FAN, FAN, a.shape[1])
    return jnp.concatenate([a4[:, r:, :], a4[:, :r, :]],
                           axis=1).reshape(a.shape)


def _bu_body(hc_ref, hp_ref, pe_ref, qw_ref, kw_ref, vw_ref, sibwo_ref,
             kpw_ref, vpw_ref, qpw_ref, parwo_ref, qb_ref, kb_ref, vb_ref,
             kpb_ref, vpb_ref, qpb_ref, lng_ref, lnb_ref, fw1_ref, fb1_ref,
             fw2_ref, fb2_ref, out_ref):
    bp = hp_ref.shape[0]
    c4 = FAN * bp
    f32 = jnp.float32
    x = hc_ref[...]
    q = x @ qw_ref[...] + qb_ref[...]
    k = x @ kw_ref[...] + kb_ref[...]
    v = x @ vw_ref[...] + vb_ref[...]
    sh, sht = _head_mats()
    pe = pe_ref[...]
    lng = lng_ref[...]
    lnb = lnb_ref[...]
    q3 = q.reshape(bp, FAN, D)
    k3 = k.reshape(bp, FAN, D)
    v3 = v.reshape(bp, FAN, D)
    x3 = x.reshape(bp, FAN, D)
    x2 = []
    for i in range(FAN):
        qi = q3[:, i, :]
        s = [jnp.dot(qi * k3[:, j, :], sh, preferred_element_type=f32)
             * SCALE + pe[i * FAN + j:i * FAN + j + 1, :] for j in range(FAN)]
        m = jnp.maximum(jnp.maximum(s[0], s[1]), jnp.maximum(s[2], s[3]))
        e = [jnp.exp(sj - m) for sj in s]
        den = e[0] + e[1] + e[2] + e[3]
        oi = sum(jnp.dot(e[j] / den, sht, preferred_element_type=f32)
                 * v3[:, j, :] for j in range(FAN))
        sai = oi @ sibwo_ref[...]
        x2.append(_ln(sai + x3[:, i, :], lng, lnb))
    nk = [x2i @ kpw_ref[...] + kpb_ref[...] for x2i in x2]
    nv = [x2i @ vpw_ref[...] + vpb_ref[...] for x2i in x2]
    pq = hp_ref[...] @ qpw_ref[...] + qpb_ref[...]
    t = [jnp.dot(pq * nk[j], sh, preferred_element_type=f32) * SCALE
         for j in range(FAN)]
    m = jnp.maximum(jnp.maximum(t[0], t[1]), jnp.maximum(t[2], t[3]))
    e = [jnp.exp(tj - m) for tj in t]
    den = e[0] + e[1] + e[2] + e[3]
    po = sum(jnp.dot(e[j] / den, sht, preferred_element_type=f32) * nv[j]
             for j in range(FAN))
    pc = po @ parwo_ref[...]
    pc = _ln(pc + pq, lng, lnb)
    hp_new = _ffn(pc, fw1_ref[...], fb1_ref[...], fw2_ref[...], fb2_ref[...])
    out_ref[...] = _ln(hp_new, lng, lnb)


def _td_body(hp_ref, hc_ref, tdg_ref, tdb_ref, tw1_ref, tb1_ref, tw2_ref,
             tb2_ref, out_ref):
    bp = hp_ref.shape[0]
    hp = hp_ref[...]
    x = hc_ref[...]
    par = jnp.broadcast_to(hp[:, None, :], (bp, FAN, D)).reshape(FAN * bp, D)
    cc = _ln(par + x, tdg_ref[...], tdb_ref[...])
    hc = _ffn(cc, tw1_ref[...], tb1_ref[...], tw2_ref[...], tb2_ref[...])
    out_ref[...] = _ln(hc, tdg_ref[...], tdb_ref[...])


def _pool_body(hb_ref, gw_ref, gb_ref, cw_ref, cb_ref, out_ref,
               m_ref, s_ref, acc_ref):
    i = pl.program_id(0)
    nb = pl.num_programs(0)
    b = hb_ref.shape[0]

    @pl.when(i == 0)
    def _():
        m_ref[0, 0] = -1e30
        s_ref[0, 0] = 0.0
        acc_ref[...] = jnp.zeros_like(acc_ref)

    hb = hb_ref[...]
    z = jnp.dot(hb, gw_ref[...], preferred_element_type=jnp.float32) + gb_ref[0, 0]
    row = i * b + lax.broadcasted_iota(jnp.int32, (b, 1), 0)
    z = jnp.where(row < N, z, -1e30)
    mb = jnp.max(z)
    m_old = m_ref[0, 0]
    m_new = jnp.maximum(m_old, mb)
    c = jnp.exp(m_old - m_new)
    e = jnp.exp(z - m_new)
    m_ref[0, 0] = m_new
    s_ref[0, 0] = s_ref[0, 0] * c + jnp.sum(e)
    acc_ref[...] = acc_ref[...] * c + jnp.sum(e * hb, axis=0, keepdims=True)

    @pl.when(i == nb - 1)
    def _():
        pooled = acc_ref[...] / s_ref[0, 0]
        out_ref[...] = jnp.dot(pooled, cw_ref[...],
                               preferred_element_type=jnp.float32) + cb_ref[...]


def kernel(node_types, emb, pe, Uq, Uk, qw, kw, vw, qpw, kpw, vpw, sibWo,
           parWo, qb, kb, vb, qpb, kpb, vpb, lng, lnb, fw1, fb1, fw2, fb2,
           tdg, tdb, tw1, tb1, tw2, tb2, gw, gb, cw, cb):
    f32 = jnp.float32
    r2 = lambda a: a.reshape(1, -1).astype(f32)
    qb2, kb2, vb2, qpb2, kpb2, vpb2 = map(r2, (qb, kb, vb, qpb, kpb, vpb))
    lng2, lnb2, fb12, fb22 = map(r2, (lng, lnb, fb1, fb2))
    tdg2, tdb2, tb12, tb22 = map(r2, (tdg, tdb, tb1, tb2))
    gb2, cb2 = map(r2, (gb, cb))

    pe_bias = pl.pallas_call(
        _pe_body,
        out_shape=jax.ShapeDtypeStruct((FAN * FAN, H), f32),
    )(pe[:FAN].astype(f32), Uq, Uk)

    par = pltpu.CompilerParams(dimension_semantics=("parallel",))
    idx = jnp.pad(node_types.astype(jnp.int32), (0, NPAD - N)).reshape(NPAD, 1)
    emb32 = emb.astype(f32)
    ehi = emb32.astype(jnp.bfloat16)
    elo = (emb32 - ehi.astype(f32)).astype(jnp.bfloat16)
    h0 = pl.pallas_call(
        _embed_body,
        grid=(NPAD // EMB_BLOCK,),
        in_specs=[pl.BlockSpec((EMB_BLOCK, 1), lambda i: (i, 0)),
                  pl.BlockSpec((VOCAB, D), lambda i: (0, 0)),
                  pl.BlockSpec((VOCAB, D), lambda i: (0, 0))],
        out_specs=pl.BlockSpec((EMB_BLOCK, D), lambda i: (i, 0)),
        out_shape=jax.ShapeDtypeStruct((NPAD, D), f32),
        compiler_params=par,
    )(idx, ehi, elo)
    hs = [h0[LEVEL_START[l]:LEVEL_START[l + 1]] for l in range(LEVELS)]

    bu_w = (qw, kw, vw, sibWo, kpw, vpw, qpw, parWo, qb2, kb2, vb2, kpb2,
            vpb2, qpb2, lng2, lnb2, fw1, fb12, fw2, fb22)
    for l in range(LEVELS - 2, -1, -1):
        p = 4 ** l
        bp = min(p, BU_BP)
        specs = [pl.BlockSpec((FAN * bp, D), lambda i: (i, 0)),
                 pl.BlockSpec((bp, D), lambda i: (i, 0)),
                 pl.BlockSpec((FAN * FAN, H), lambda i: (0, 0))]
        specs += [pl.BlockSpec(w.shape, lambda i: (0, 0)) for w in bu_w]
        hs[l] = pl.pallas_call(
            _bu_body,
            grid=(p // bp,),
            in_specs=specs,
            out_specs=pl.BlockSpec((bp, D), lambda i: (i, 0)),
            out_shape=jax.ShapeDtypeStruct((p, D), f32),
            compiler_params=par,
        )(hs[l + 1], hs[l], pe_bias, *bu_w)

    td_w = (tdg2, tdb2, tw1, tb12, tw2, tb22)
    for l in range(LEVELS - 1):
        p = 4 ** l
        bp = min(p, TD_BP)
        specs = [pl.BlockSpec((bp, D), lambda i: (i, 0)),
                 pl.BlockSpec((FAN * bp, D), lambda i: (i, 0))]
        specs += [pl.BlockSpec(w.shape, lambda i: (0, 0)) for w in td_w]
        hs[l + 1] = pl.pallas_call(
            _td_body,
            grid=(p // bp,),
            in_specs=specs,
            out_specs=pl.BlockSpec((FAN * bp, D), lambda i: (i, 0)),
            out_shape=jax.ShapeDtypeStruct((FAN * p, D), f32),
            compiler_params=par,
        )(hs[l], hs[l + 1], *td_w)

    hbig = jnp.pad(jnp.concatenate(hs, axis=0), ((0, NPAD - N), (0, 0)))
    out = pl.pallas_call(
        _pool_body,
        grid=(NPAD // POOL_BLOCK,),
        in_specs=[pl.BlockSpec((POOL_BLOCK, D), lambda i: (i, 0)),
                  pl.BlockSpec((D, 1), lambda i: (0, 0)),
                  pl.BlockSpec((1, 1), lambda i: (0, 0)),
                  pl.BlockSpec((D, NCLS), lambda i: (0, 0)),
                  pl.BlockSpec((1, NCLS), lambda i: (0, 0))],
        out_specs=pl.BlockSpec((1, NCLS), lambda i: (0, 0)),
        out_shape=jax.ShapeDtypeStruct((1, NCLS), f32),
        scratch_shapes=[pltpu.SMEM((1, 1), f32), pltpu.SMEM((1, 1), f32),
                        pltpu.VMEM((1, D), f32)],
    )(hbig, gw.astype(f32), gb2, cw.astype(f32), cb2)
    return out
